# baseline (speedup 1.0000x reference)
import jax
import jax.numpy as jnp
from jax import lax
from jax.experimental import pallas as pl
from jax.experimental.pallas import tpu as pltpu
from jax.experimental.pallas import tpu_sc as plsc

VOCAB = 100000
HIDDEN = 128
MAXLEN = 2048
BATCH = 4
SEQ = 2048

NC = 2
NS = 16
NW = NC * NS
ROWS = BATCH * SEQ
RPW = ROWS // NW
GCHUNK = 128
NG = RPW // GCHUNK
NCH = HIDDEN // 16


def _body(x_hbm, segm_hbm, tok_hbm, segtab_hbm, pos_hbm, out_hbm,
          idx_v, segm_v, tok_v, pos_v, segtab_v, sem_t):
    wid = lax.axis_index("s") * NC + lax.axis_index("c")
    base = wid * RPW
    pos_base = lax.rem(base, SEQ)

    pltpu.sync_copy(x_hbm.at[wid], idx_v)

    copies = []
    for j in range(NG):
        copies.append(pltpu.async_copy(tok_hbm.at[idx_v.at[j]],
                                       tok_v.at[pl.ds(j * GCHUNK, GCHUNK)],
                                       sem_t))

    pltpu.sync_copy(segm_hbm.at[wid], segm_v)
    pltpu.sync_copy(segtab_hbm, segtab_v)
    pltpu.sync_copy(pos_hbm.at[pl.ds(pos_base, RPW)], pos_v)

    for c in copies:
        c.wait()

    seg0 = [segtab_v[0, pl.ds(c * 16, 16)] for c in range(NCH)]
    diff = [segtab_v[1, pl.ds(c * 16, 16)] - seg0[c] for c in range(NCH)]

    def add_body(r, carry):
        mv = segm_v[r, :]
        for c in range(NCH):
            sl = pl.ds(c * 16, 16)
            tok_v[r, sl] = (tok_v[r, sl] + pos_v[r, sl]
                            + (seg0[c] + mv * diff[c]))
        return carry

    lax.fori_loop(0, RPW, add_body, 0)

    pltpu.sync_copy(tok_v, out_hbm.at[pl.ds(base, RPW)])


@jax.jit
def _run(x3, segm, tok_table, seg_table, pos_table):
    mesh = plsc.VectorSubcoreMesh(core_axis_name="c", subcore_axis_name="s",
                                  num_cores=NC, num_subcores=NS)
    fn = pl.kernel(
        _body,
        out_type=jax.ShapeDtypeStruct((ROWS, HIDDEN), jnp.float32),
        mesh=mesh,
        scratch_types=[
            pltpu.VMEM((NG, GCHUNK), jnp.int32),
            pltpu.VMEM((RPW, 16), jnp.float32),
            pltpu.VMEM((RPW, HIDDEN), jnp.float32),
            pltpu.VMEM((RPW, HIDDEN), jnp.float32),
            pltpu.VMEM((2, HIDDEN), jnp.float32),
            pltpu.SemaphoreType.DMA,
        ],
    )
    return fn(x3, segm, tok_table, seg_table, pos_table)


def kernel(x, segment_ids, tok_table, seg_table, pos_table):
    x3 = x.reshape(NW, NG, GCHUNK).astype(jnp.int32)
    segm = jnp.broadcast_to(
        segment_ids.reshape(NW, RPW, 1).astype(jnp.float32), (NW, RPW, 16))
    out = _run(x3, segm, tok_table, seg_table, pos_table)
    return out.reshape(BATCH, SEQ, HIDDEN)
